# baseline (speedup 1.0000x reference)
import jax
import jax.numpy as jnp
from jax.experimental import pallas as pl

STIM_SCALE = 1.0 / (0.0066 * 2)
N, E, IN, H, OUT, EF = 20000, 4096, 6, 64, 1, 2


def _edge_feats(stim):
    s = (stim * STIM_SCALE)[:, None]
    return jnp.concatenate([s, jnp.ones_like(s)], axis=-1)


def _nnconv(x, edge_index, stim, Wa, ba, Wb, bb, bias, fi, fo):
    n = x.shape[0]
    ef = _edge_feats(stim)
    h = jax.nn.relu(ef @ Wa + ba)
    We = (h @ Wb + bb).reshape(-1, fi, fo)
    src, dst = edge_index[0], edge_index[1]
    m = jnp.einsum('ei,eio->eo', x[src], We)
    s = jax.ops.segment_sum(m, dst, num_segments=n)
    c = jax.ops.segment_sum(jnp.ones((m.shape[0], 1), m.dtype), dst, num_segments=n)
    return s / jnp.maximum(c, 1.0) + bias


def _ln(x, g, b):
    mu = jnp.mean(x, -1, keepdims=True)
    v = jnp.mean((x - mu) ** 2, -1, keepdims=True)
    return (x - mu) / jnp.sqrt(v + 1e-5) * g + b


def _copy_kernel(x_ref, o_ref):
    o_ref[...] = x_ref[...]


def kernel(x, edge_index1, stim1, edge_index2, stim2, edge_index3, stim3, W1a, b1a, W1b, b1b, bc1, W2a, b2a, W2b, b2b, bc2, W3a, b3a, W3b, b3b, bc3, g1, be1, g2, be2):
    h = jax.nn.relu(_ln(_nnconv(x, edge_index1, stim1, W1a, b1a, W1b, b1b, bc1, IN, H), g1, be1))
    h = jax.nn.relu(_ln(_nnconv(h, edge_index2, stim2, W2a, b2a, W2b, b2b, bc2, H, H), g2, be2))
    out = jax.nn.softplus(_nnconv(h, edge_index3, stim3, W3a, b3a, W3b, b3b, bc3, H, OUT))
    out = pl.pallas_call(
        _copy_kernel,
        out_shape=jax.ShapeDtypeStruct(out.shape, out.dtype),
    )(out)
    return out


# speedup vs baseline: 1.2055x; 1.2055x over previous
import functools

import jax
import jax.numpy as jnp
from jax import lax
from jax.experimental import pallas as pl
from jax.experimental.pallas import tpu as pltpu
from jax.experimental.pallas import tpu_sc as plsc

STIM_SCALE = 1.0 / (0.0066 * 2)
N, E, IN, H, OUT = 20000, 4096, 6, 64, 1
NC, NS = 2, 16
PW = 64
CW = 16
FW = 16
NF = 20480
ZJ = E + E // 2
ZJT = ZJ // NS
EJT = (E // NC) // NS
GJT = E // NS
NFT = NF // NS

_mesh = plsc.VectorSubcoreMesh(core_axis_name="c", subcore_axis_name="s")
_sc_params = pltpu.CompilerParams(use_tc_tiling_on_sc=False)


def _sc_gather_rows(table, idx, width):
    nrow = idx.shape[0]
    per = nrow // (NC * NS)

    @functools.partial(
        pl.kernel,
        out_type=jax.ShapeDtypeStruct((nrow, width), jnp.float32),
        mesh=_mesh,
        compiler_params=_sc_params,
        scratch_types=[pltpu.VMEM((per,), jnp.int32),
                       pltpu.VMEM((per, width), jnp.float32)],
    )
    def k(t_hbm, i_hbm, o_hbm, idx_v, rows_v):
        wid = lax.axis_index("s") * NC + lax.axis_index("c")
        base = wid * per
        pltpu.sync_copy(i_hbm.at[pl.ds(base, per)], idx_v)
        pltpu.sync_copy(t_hbm.at[idx_v], rows_v)
        pltpu.sync_copy(rows_v, o_hbm.at[pl.ds(base, per)])

    return k(table, idx)


def _sc_join(scat, dst, zidx, gidx, zeros_j, width):
    @functools.partial(
        pl.kernel,
        out_type=jax.ShapeDtypeStruct((NC, E, width), jnp.float32),
        mesh=_mesh,
        compiler_params=_sc_params,
        scratch_types=[pltpu.VMEM_SHARED((N, width), jnp.float32),
                       pltpu.VMEM((ZJT, width), jnp.float32),
                       pltpu.VMEM((ZJT,), jnp.int32),
                       pltpu.VMEM((EJT, width), jnp.float32),
                       pltpu.VMEM((EJT,), jnp.int32),
                       pltpu.VMEM((GJT, width), jnp.float32),
                       pltpu.VMEM((GJT,), jnp.int32)],
    )
    def k(s_hbm, d_hbm, z_hbm, g_hbm, zero_hbm, o_hbm,
          acc, zbuf, zidx_v, rows_v, didx_v, gout_v, gidx_v):
        c = lax.axis_index("c")
        t = lax.axis_index("s")
        pltpu.sync_copy(zero_hbm, zbuf)
        pltpu.sync_copy(z_hbm.at[c, pl.ds(t * ZJT, ZJT)], zidx_v)
        pltpu.sync_copy(zbuf, acc.at[zidx_v])
        plsc.subcore_barrier()
        off = c * (E // NC) + t * EJT
        pltpu.sync_copy(s_hbm.at[pl.ds(off, EJT)], rows_v)
        pltpu.sync_copy(d_hbm.at[pl.ds(off, EJT)], didx_v)
        pltpu.sync_copy(rows_v, acc.at[didx_v], add=True)
        plsc.subcore_barrier()
        pltpu.sync_copy(g_hbm.at[pl.ds(t * GJT, GJT)], gidx_v)
        pltpu.sync_copy(acc.at[gidx_v], gout_v)
        pltpu.sync_copy(gout_v, o_hbm.at[c, pl.ds(t * GJT, GJT)])

    return k(scat, dst, zidx, gidx, zeros_j)


def _sc_final_scatter(scat3, dst3, zeros_f):
    @functools.partial(
        pl.kernel,
        out_type=jax.ShapeDtypeStruct((NC, NF, FW), jnp.float32),
        mesh=_mesh,
        compiler_params=_sc_params,
        scratch_types=[pltpu.VMEM_SHARED((NF, FW), jnp.float32),
                       pltpu.VMEM((NFT, FW), jnp.float32),
                       pltpu.VMEM((EJT, FW), jnp.float32),
                       pltpu.VMEM((EJT,), jnp.int32)],
    )
    def k(s_hbm, d_hbm, zero_hbm, o_hbm, acc, zbuf, rows_v, didx_v):
        c = lax.axis_index("c")
        t = lax.axis_index("s")
        pltpu.sync_copy(zero_hbm, zbuf)
        pltpu.sync_copy(zbuf, acc.at[pl.ds(t * NFT, NFT)])
        plsc.subcore_barrier()
        off = c * (E // NC) + t * EJT
        pltpu.sync_copy(s_hbm.at[pl.ds(off, EJT)], rows_v)
        pltpu.sync_copy(d_hbm.at[pl.ds(off, EJT)], didx_v)
        pltpu.sync_copy(rows_v, acc.at[didx_v], add=True)
        plsc.subcore_barrier()
        pltpu.sync_copy(acc.at[pl.ds(t * NFT, NFT)], zbuf)
        pltpu.sync_copy(zbuf, o_hbm.at[c, pl.ds(t * NFT, NFT)])

    return k(scat3, dst3, zeros_f)


def _k1_body(stim_ref, wa_ref, w1b_ref, b1b_ref, xg_ref, o_ref):
    s = stim_ref[...] * STIM_SCALE
    wa = wa_ref[...]
    h = jnp.maximum(s * wa[0:1, :] + wa[1:2, :], 0.0)
    we = jnp.dot(h, w1b_ref[...], preferred_element_type=jnp.float32,
                 precision=jax.lax.Precision.HIGHEST)
    we = we + b1b_ref[...].reshape(1, -1)
    xg = xg_ref[...]
    m = jnp.zeros((E, H), jnp.float32)
    for i in range(IN):
        m = m + xg[:, i:i + 1] * we[:, i * H:(i + 1) * H]
    o_ref[...] = m


def _ln_relu(pre, g, b):
    mu = jnp.mean(pre, axis=-1, keepdims=True)
    var = jnp.mean((pre - mu) ** 2, axis=-1, keepdims=True)
    return jnp.maximum((pre - mu) / jnp.sqrt(var + 1e-5) * g.reshape(1, -1) + b.reshape(1, -1), 0.0)


EB, KB = 512, 512
NEB, NKB = E // EB, (H * H) // KB


def _k2_body(stim_ref, wa_ref, w2b_ref, b2b_ref, partm_ref, partc_ref,
             bc1_ref, g1_ref, be1_ref, o_ref, acc_ref):
    kb = pl.program_id(1)
    s = stim_ref[...] * STIM_SCALE
    wa = wa_ref[...]
    r = jnp.maximum(s * wa[0:1, :] + wa[1:2, :], 0.0)
    prod = jnp.dot(r.astype(jnp.bfloat16), w2b_ref[...],
                   preferred_element_type=jnp.float32)

    @pl.when(kb == 0)
    def _():
        acc_ref[...] = prod

    @pl.when(kb > 0)
    def _():
        acc_ref[...] += prod

    @pl.when(kb == NKB - 1)
    def _():
        we = acc_ref[...] + b2b_ref[...].reshape(1, -1)
        agg = partm_ref[0] + partm_ref[1]
        cnt = partc_ref[0, :, 0:1] + partc_ref[1, :, 0:1]
        pre = agg / jnp.maximum(cnt, 1.0) + bc1_ref[...].reshape(1, -1)
        h1 = _ln_relu(pre, g1_ref[...], be1_ref[...])
        m = jnp.zeros((EB, H), jnp.float32)
        for kk in range(H):
            m = m + h1[:, kk:kk + 1] * we[:, kk * H:(kk + 1) * H]
        o_ref[...] = m


def _k3_body(stim_ref, wa_ref, w3b_ref, b3b_ref, partm_ref, partc_ref,
             bc2_ref, g2_ref, be2_ref, o_ref):
    agg = partm_ref[0] + partm_ref[1]
    cnt = partc_ref[0, :, 0:1] + partc_ref[1, :, 0:1]
    pre = agg / jnp.maximum(cnt, 1.0) + bc2_ref[...].reshape(1, -1)
    h2 = _ln_relu(pre, g2_ref[...], be2_ref[...])
    s = stim_ref[...] * STIM_SCALE
    wa = wa_ref[...]
    r = jnp.maximum(s * wa[0:1, :] + wa[1:2, :], 0.0)
    we = jnp.dot(r, w3b_ref[...], preferred_element_type=jnp.float32,
                 precision=jax.lax.Precision.HIGHEST)
    we = we + b3b_ref[...].reshape(1, -1)
    m3 = jnp.sum(h2 * we, axis=1, keepdims=True)
    o_ref[...] = jnp.concatenate(
        [m3, jnp.ones((E, 1), jnp.float32), jnp.zeros((E, FW - 2), jnp.float32)], axis=1)


def _kf_body(acc_ref, bc3_ref, o_ref):
    acc = acc_ref[0] + acc_ref[1]
    val = acc[:N, 0:1]
    cnt = acc[:N, 1:2]
    pre = val / jnp.maximum(cnt, 1.0) + bc3_ref[...].reshape(1, 1)
    o_ref[...] = jnp.maximum(pre, 0.0) + jnp.log1p(jnp.exp(-jnp.abs(pre)))


def _full(shape):
    return pl.BlockSpec(shape, lambda *_: tuple(0 for _ in shape))


def kernel(x, edge_index1, stim1, edge_index2, stim2, edge_index3, stim3,
           W1a, b1a, W1b, b1b, bc1, W2a, b2a, W2b, b2b, bc2,
           W3a, b3a, W3b, b3b, bc3, g1, be1, g2, be2):
    f32 = jnp.float32
    src1, dst1 = edge_index1[0], edge_index1[1]
    src2, dst2 = edge_index2[0], edge_index2[1]
    src3, dst3 = edge_index3[0], edge_index3[1]

    xpad = jnp.concatenate([x, jnp.zeros((N, 16 - IN), f32)], axis=1)
    w1ac = jnp.stack([W1a[0], W1a[1] + b1a])
    w2ac = jnp.stack([W2a[0], W2a[1] + b2a])
    w3ac = jnp.stack([W3a[0], W3a[1] + b3a])
    w2b_bf = W2b.astype(jnp.bfloat16)
    half = E // NC
    zidx12 = jnp.stack([jnp.concatenate([src2, dst1[:half]]),
                        jnp.concatenate([src2, dst1[half:]])])
    zidx23 = jnp.stack([jnp.concatenate([src3, dst2[:half]]),
                        jnp.concatenate([src3, dst2[half:]])])
    zeros_jm = jnp.zeros((ZJT, PW), f32)
    zeros_jc = jnp.zeros((ZJT, CW), f32)
    zeros_f = jnp.zeros((NFT, FW), f32)
    ones_c = jnp.concatenate([jnp.ones((E, 1), f32), jnp.zeros((E, CW - 1), f32)], axis=1)
    s1 = stim1.reshape(E, 1)
    s2 = stim2.reshape(E, 1)
    s3 = stim3.reshape(E, 1)

    xg = _sc_gather_rows(xpad, src1, 16)
    scat1 = pl.pallas_call(
        _k1_body,
        grid=(1,),
        in_specs=[_full((E, 1)), _full((2, IN * H)), _full((IN * H, IN * H)),
                  _full((IN * H,)), _full((E, 16))],
        out_specs=_full((E, PW)),
        out_shape=jax.ShapeDtypeStruct((E, PW), f32),
    )(s1, w1ac, W1b, b1b, xg)

    part12m = _sc_join(scat1, dst1, zidx12, src2, zeros_jm, PW)
    part12c = _sc_join(ones_c, dst1, zidx12, src2, zeros_jc, CW)

    scat2 = pl.pallas_call(
        _k2_body,
        grid=(NEB, NKB),
        in_specs=[
            pl.BlockSpec((EB, 1), lambda eb, kb: (eb, 0)),
            pl.BlockSpec((2, KB), lambda eb, kb: (0, kb)),
            pl.BlockSpec((KB, H * H), lambda eb, kb: (kb, 0)),
            pl.BlockSpec((H * H,), lambda eb, kb: (0,)),
            pl.BlockSpec((NC, EB, PW), lambda eb, kb: (0, eb, 0)),
            pl.BlockSpec((NC, EB, CW), lambda eb, kb: (0, eb, 0)),
            pl.BlockSpec((H,), lambda eb, kb: (0,)),
            pl.BlockSpec((H,), lambda eb, kb: (0,)),
            pl.BlockSpec((H,), lambda eb, kb: (0,)),
        ],
        out_specs=pl.BlockSpec((EB, PW), lambda eb, kb: (eb, 0)),
        out_shape=jax.ShapeDtypeStruct((E, PW), f32),
        scratch_shapes=[pltpu.VMEM((EB, H * H), f32)],
    )(s2, w2ac, w2b_bf, b2b, part12m, part12c, bc1, g1, be1)

    part23m = _sc_join(scat2, dst2, zidx23, src3, zeros_jm, PW)
    part23c = _sc_join(ones_c, dst2, zidx23, src3, zeros_jc, CW)

    scat3 = pl.pallas_call(
        _k3_body,
        grid=(1,),
        in_specs=[_full((E, 1)), _full((2, H)), _full((H, H)), _full((H,)),
                  _full((NC, E, PW)), _full((NC, E, CW)),
                  _full((H,)), _full((H,)), _full((H,))],
        out_specs=_full((E, FW)),
        out_shape=jax.ShapeDtypeStruct((E, FW), f32),
    )(s3, w3ac, W3b, b3b, part23m, part23c, bc2, g2, be2)

    facc = _sc_final_scatter(scat3, dst3, zeros_f)

    out = pl.pallas_call(
        _kf_body,
        grid=(1,),
        in_specs=[_full((NC, NF, FW)), _full((OUT,))],
        out_specs=_full((N, OUT)),
        out_shape=jax.ShapeDtypeStruct((N, OUT), f32),
    )(facc, bc3)
    return out


# speedup vs baseline: 1.6645x; 1.3807x over previous
import functools

import jax
import jax.numpy as jnp
from jax import lax
from jax.experimental import pallas as pl
from jax.experimental.pallas import tpu as pltpu
from jax.experimental.pallas import tpu_sc as plsc

STIM_SCALE = 1.0 / (0.0066 * 2)
N, E, IN, H, OUT = 20000, 4096, 6, 64, 1
NC, NS = 2, 16
PW = 64
CW = 16
FW = 16
NF = 20480
ZJ = E + E // 2
ZJT = ZJ // NS
EJT = (E // NC) // NS
GJT = E // NS
NFT = NF // NS

_mesh = plsc.VectorSubcoreMesh(core_axis_name="c", subcore_axis_name="s")
_sc_params = pltpu.CompilerParams(use_tc_tiling_on_sc=False)


def _sc_gather_rows(table, idx, width):
    nrow = idx.shape[0]
    per = nrow // (NC * NS)

    @functools.partial(
        pl.kernel,
        out_type=jax.ShapeDtypeStruct((nrow, width), jnp.float32),
        mesh=_mesh,
        compiler_params=_sc_params,
        scratch_types=[pltpu.VMEM((per,), jnp.int32),
                       pltpu.VMEM((per, width), jnp.float32)],
    )
    def k(t_hbm, i_hbm, o_hbm, idx_v, rows_v):
        wid = lax.axis_index("s") * NC + lax.axis_index("c")
        base = wid * per
        pltpu.sync_copy(i_hbm.at[pl.ds(base, per)], idx_v)
        pltpu.sync_copy(t_hbm.at[idx_v], rows_v)
        pltpu.sync_copy(rows_v, o_hbm.at[pl.ds(base, per)])

    return k(table, idx)


def _sc_join(scat, dst, zidx, gidx, zeros_j, width):
    @functools.partial(
        pl.kernel,
        out_type=jax.ShapeDtypeStruct((NC, E, width), jnp.float32),
        mesh=_mesh,
        compiler_params=_sc_params,
        scratch_types=[pltpu.VMEM_SHARED((N, width), jnp.float32),
                       pltpu.VMEM((ZJT, width), jnp.float32),
                       pltpu.VMEM((ZJT,), jnp.int32),
                       pltpu.VMEM((EJT, width), jnp.float32),
                       pltpu.VMEM((EJT,), jnp.int32),
                       pltpu.VMEM((GJT, width), jnp.float32),
                       pltpu.VMEM((GJT,), jnp.int32)],
    )
    def k(s_hbm, d_hbm, z_hbm, g_hbm, zero_hbm, o_hbm,
          acc, zbuf, zidx_v, rows_v, didx_v, gout_v, gidx_v):
        c = lax.axis_index("c")
        t = lax.axis_index("s")
        pltpu.sync_copy(zero_hbm, zbuf)
        pltpu.sync_copy(z_hbm.at[c, pl.ds(t * ZJT, ZJT)], zidx_v)
        pltpu.sync_copy(zbuf, acc.at[zidx_v])
        plsc.subcore_barrier()
        off = c * (E // NC) + t * EJT
        pltpu.sync_copy(s_hbm.at[pl.ds(off, EJT)], rows_v)
        pltpu.sync_copy(d_hbm.at[pl.ds(off, EJT)], didx_v)
        pltpu.sync_copy(rows_v, acc.at[didx_v], add=True)
        plsc.subcore_barrier()
        pltpu.sync_copy(g_hbm.at[pl.ds(t * GJT, GJT)], gidx_v)
        pltpu.sync_copy(acc.at[gidx_v], gout_v)
        pltpu.sync_copy(gout_v, o_hbm.at[c, pl.ds(t * GJT, GJT)])

    return k(scat, dst, zidx, gidx, zeros_j)


def _sc_final_scatter(scat3, dst3, zeros_f):
    @functools.partial(
        pl.kernel,
        out_type=jax.ShapeDtypeStruct((NC, NF, FW), jnp.float32),
        mesh=_mesh,
        compiler_params=_sc_params,
        scratch_types=[pltpu.VMEM_SHARED((NF, FW), jnp.float32),
                       pltpu.VMEM((NFT, FW), jnp.float32),
                       pltpu.VMEM((EJT, FW), jnp.float32),
                       pltpu.VMEM((EJT,), jnp.int32)],
    )
    def k(s_hbm, d_hbm, zero_hbm, o_hbm, acc, zbuf, rows_v, didx_v):
        c = lax.axis_index("c")
        t = lax.axis_index("s")
        pltpu.sync_copy(zero_hbm, zbuf)
        pltpu.sync_copy(zbuf, acc.at[pl.ds(t * NFT, NFT)])
        plsc.subcore_barrier()
        off = c * (E // NC) + t * EJT
        pltpu.sync_copy(s_hbm.at[pl.ds(off, EJT)], rows_v)
        pltpu.sync_copy(d_hbm.at[pl.ds(off, EJT)], didx_v)
        pltpu.sync_copy(rows_v, acc.at[didx_v], add=True)
        plsc.subcore_barrier()
        pltpu.sync_copy(acc.at[pl.ds(t * NFT, NFT)], zbuf)
        pltpu.sync_copy(zbuf, o_hbm.at[c, pl.ds(t * NFT, NFT)])

    return k(scat3, dst3, zeros_f)


def _b(x):
    return x.astype(jnp.bfloat16).astype(jnp.float32)


def _dot3(a, b):
    a_hi = a.astype(jnp.bfloat16)
    a_lo = (a - a_hi.astype(jnp.float32)).astype(jnp.bfloat16)
    b_hi = b.astype(jnp.bfloat16)
    b_lo = (b - b_hi.astype(jnp.float32)).astype(jnp.bfloat16)
    f32 = jnp.float32
    return (jnp.dot(a_hi, b_hi, preferred_element_type=f32)
            + jnp.dot(a_lo, b_hi, preferred_element_type=f32)
            + jnp.dot(a_hi, b_lo, preferred_element_type=f32))


def _k1_body(stim_ref, wa_ref, w1b_ref, b1b_ref, xg_ref, o_ref):
    s = _b(stim_ref[...] * STIM_SCALE)
    wa = wa_ref[...]
    h = jnp.maximum(s * wa[0:1, :] + wa[1:2, :], 0.0)
    we = jnp.dot(h.astype(jnp.bfloat16), w1b_ref[...].astype(jnp.bfloat16),
                 preferred_element_type=jnp.float32)
    we = we + b1b_ref[...].reshape(1, -1)
    xg = xg_ref[...]
    m = jnp.zeros((E, H), jnp.float32)
    for i in range(IN):
        m = m + _b(xg[:, i:i + 1]) * _b(we[:, i * H:(i + 1) * H])
    o_ref[...] = m


def _ln_relu(pre, g, b):
    mu = jnp.mean(pre, axis=-1, keepdims=True)
    var = jnp.mean((pre - mu) ** 2, axis=-1, keepdims=True)
    return jnp.maximum((pre - mu) / jnp.sqrt(var + 1e-5) * g.reshape(1, -1) + b.reshape(1, -1), 0.0)


EB = 128
NEB = E // EB
KR = (H * H) // 8


def _h1_block(partm_ref, partc_ref, bc1_ref, g1_ref, be1_ref):
    agg = partm_ref[0] + partm_ref[1]
    cnt = partc_ref[0, :, 0:1] + partc_ref[1, :, 0:1]
    pre = agg / jnp.maximum(cnt, 1.0) + bc1_ref[...].reshape(1, -1)
    return _ln_relu(pre, g1_ref[...], be1_ref[...])


def _edge_contract(h1, we):
    m = jnp.zeros((h1.shape[0], H), jnp.float32)
    for kk in range(H):
        m = m + h1[:, kk:kk + 1] * we[:, kk * H:(kk + 1) * H]
    return m


PC = 512
NPC = (H * H) // PC
KPC = PC // H


def _k2_body(stim_ref, wa_ref, w2b_ref, b2b_ref,
             partm_ref, partc_ref, bc1_ref, g1_ref, be1_ref, o_ref):
    s = _b(stim_ref[...] * STIM_SCALE)
    wa = wa_ref[...]
    r = jnp.maximum(s * wa[0:1, :] + wa[1:2, :], 0.0)
    r_b = r.astype(jnp.bfloat16)
    h1 = _h1_block(partm_ref, partc_ref, bc1_ref, g1_ref, be1_ref)
    m = jnp.zeros((EB, H), jnp.float32)
    for pc in range(NPC):
        wec = jnp.dot(r_b, w2b_ref[:, pl.ds(pc * PC, PC)],
                      preferred_element_type=jnp.float32)
        wec = wec + b2b_ref[pl.ds(pc * PC, PC)].reshape(1, -1)
        for kk in range(KPC):
            k = pc * KPC + kk
            m = m + _b(h1[:, k:k + 1]) * _b(wec[:, kk * H:(kk + 1) * H])
    o_ref[...] = m


def _k3_body(stim_ref, wa_ref, w3b_ref, b3b_ref, partm_ref, partc_ref,
             bc2_ref, g2_ref, be2_ref, o_ref):
    agg = partm_ref[0] + partm_ref[1]
    cnt = partc_ref[0, :, 0:1] + partc_ref[1, :, 0:1]
    pre = agg / jnp.maximum(cnt, 1.0) + bc2_ref[...].reshape(1, -1)
    h2 = _ln_relu(pre, g2_ref[...], be2_ref[...])
    s = _b(stim_ref[...] * STIM_SCALE)
    wa = wa_ref[...]
    r = jnp.maximum(s * wa[0:1, :] + wa[1:2, :], 0.0)
    we = jnp.dot(r.astype(jnp.bfloat16), w3b_ref[...].astype(jnp.bfloat16),
                 preferred_element_type=jnp.float32)
    we = we + b3b_ref[...].reshape(1, -1)
    m3 = jnp.sum(_b(h2) * _b(we), axis=1, keepdims=True)
    o_ref[...] = jnp.concatenate(
        [m3, jnp.ones((E, 1), jnp.float32), jnp.zeros((E, FW - 2), jnp.float32)], axis=1)


def _kf_body(acc_ref, bc3_ref, o_ref):
    acc = acc_ref[0] + acc_ref[1]
    val = acc[:N, 0:1]
    cnt = acc[:N, 1:2]
    pre = val / jnp.maximum(cnt, 1.0) + bc3_ref[...].reshape(1, 1)
    o_ref[...] = jnp.maximum(pre, 0.0) + jnp.log1p(jnp.exp(-jnp.abs(pre)))


def _full(shape):
    return pl.BlockSpec(shape, lambda *_: tuple(0 for _ in shape))


def kernel(x, edge_index1, stim1, edge_index2, stim2, edge_index3, stim3,
           W1a, b1a, W1b, b1b, bc1, W2a, b2a, W2b, b2b, bc2,
           W3a, b3a, W3b, b3b, bc3, g1, be1, g2, be2):
    f32 = jnp.float32
    src1, dst1 = edge_index1[0], edge_index1[1]
    src2, dst2 = edge_index2[0], edge_index2[1]
    src3, dst3 = edge_index3[0], edge_index3[1]

    bf = jnp.bfloat16

    def _rnd(v):
        return v.astype(bf).astype(f32)

    xpad = jnp.concatenate([x, jnp.zeros((N, 128 - IN), f32)], axis=1)
    w1ac = jnp.stack([_rnd(W1a[0]), _rnd(W1a[1]) + b1a])
    w2ac = jnp.stack([_rnd(W2a[0]), _rnd(W2a[1]) + b2a])
    w3ac = jnp.stack([_rnd(W3a[0]), _rnd(W3a[1]) + b3a])
    half = E // NC
    zidx12 = jnp.stack([jnp.concatenate([src2, dst1[:half]]),
                        jnp.concatenate([src2, dst1[half:]])])
    zidx23 = jnp.stack([jnp.concatenate([src3, dst2[:half]]),
                        jnp.concatenate([src3, dst2[half:]])])
    zeros_jm = jnp.zeros((ZJT, PW), f32)
    zeros_jc = jnp.zeros((ZJT, CW), f32)
    zeros_f = jnp.zeros((NFT, FW), f32)
    ones_c = jnp.concatenate([jnp.ones((E, 1), f32), jnp.zeros((E, CW - 1), f32)], axis=1)
    s1 = stim1.reshape(E, 1)
    s2 = stim2.reshape(E, 1)
    s3 = stim3.reshape(E, 1)

    xg = _sc_gather_rows(xpad, src1, 128)
    scat1 = pl.pallas_call(
        _k1_body,
        grid=(1,),
        in_specs=[_full((E, 1)), _full((2, IN * H)), _full((IN * H, IN * H)),
                  _full((IN * H,)), _full((E, 128))],
        out_specs=_full((E, PW)),
        out_shape=jax.ShapeDtypeStruct((E, PW), f32),
    )(s1, w1ac, W1b, b1b, xg)

    part12m = _sc_join(scat1, dst1, zidx12, src2, zeros_jm, PW)
    part12c = _sc_join(ones_c, dst1, zidx12, src2, zeros_jc, CW)

    w2b_bf = W2b.astype(bf)
    scat2 = pl.pallas_call(
        _k2_body,
        grid=(NEB,),
        in_specs=[
            pl.BlockSpec((EB, 1), lambda eb: (eb, 0)),
            pl.BlockSpec((2, H * H), lambda eb: (0, 0)),
            pl.BlockSpec((H * H, H * H), lambda eb: (0, 0)),
            pl.BlockSpec((H * H,), lambda eb: (0,)),
            pl.BlockSpec((NC, EB, PW), lambda eb: (0, eb, 0)),
            pl.BlockSpec((NC, EB, CW), lambda eb: (0, eb, 0)),
            pl.BlockSpec((H,), lambda eb: (0,)),
            pl.BlockSpec((H,), lambda eb: (0,)),
            pl.BlockSpec((H,), lambda eb: (0,)),
        ],
        out_specs=pl.BlockSpec((EB, PW), lambda eb: (eb, 0)),
        out_shape=jax.ShapeDtypeStruct((E, PW), f32),
    )(s2, w2ac, w2b_bf, b2b, part12m, part12c, bc1, g1, be1)

    part23m = _sc_join(scat2, dst2, zidx23, src3, zeros_jm, PW)
    part23c = _sc_join(ones_c, dst2, zidx23, src3, zeros_jc, CW)

    scat3 = pl.pallas_call(
        _k3_body,
        grid=(1,),
        in_specs=[_full((E, 1)), _full((2, H)), _full((H, H)), _full((H,)),
                  _full((NC, E, PW)), _full((NC, E, CW)),
                  _full((H,)), _full((H,)), _full((H,))],
        out_specs=_full((E, FW)),
        out_shape=jax.ShapeDtypeStruct((E, FW), f32),
    )(s3, w3ac, W3b, b3b, part23m, part23c, bc2, g2, be2)

    facc = _sc_final_scatter(scat3, dst3, zeros_f)

    out = pl.pallas_call(
        _kf_body,
        grid=(1,),
        in_specs=[_full((NC, NF, FW)), _full((OUT,))],
        out_specs=_full((N, OUT)),
        out_shape=jax.ShapeDtypeStruct((N, OUT), f32),
    )(facc, bc3)
    return out
